# baseline (speedup 1.0000x reference)
import functools

import jax
import jax.numpy as jnp
from jax import lax
from jax.experimental import pallas as pl
from jax.experimental.pallas import tpu as pltpu
from jax.experimental.pallas import tpu_sc as plsc

SMOOTHING = 0.1
CONFIDENCE = 1.0 - SMOOTHING

BR = 8
NBUF = 8


def _sc_gather_build(B, C):
    info = plsc.get_sparse_core_info()
    nw = info.num_cores * info.num_subcores
    per_w = B // nw
    mesh = plsc.VectorSubcoreMesh(core_axis_name="c", subcore_axis_name="s")

    @functools.partial(
        pl.kernel,
        mesh=mesh,
        out_type=jax.ShapeDtypeStruct((B,), jnp.float32),
        scratch_types=[
            pltpu.VMEM((per_w,), jnp.int32),
            pltpu.VMEM((per_w,), jnp.float32),
            pltpu.SemaphoreType.DMA,
        ],
    )
    def gather_k(pred_flat_hbm, target_hbm, out_hbm, idx_v, vals_v, sem):
        wid = lax.axis_index("s") * info.num_cores + lax.axis_index("c")
        base = wid * per_w
        pltpu.sync_copy(target_hbm.at[pl.ds(base, per_w)], idx_v)
        for jj in range(per_w // 16):
            t = idx_v[pl.ds(jj * 16, 16)]
            rows = (base + jj * 16) + lax.iota(jnp.int32, 16)
            idx_v[pl.ds(jj * 16, 16)] = t + rows * C
        pltpu.async_copy(pred_flat_hbm.at[idx_v], vals_v, sem).wait()
        pltpu.sync_copy(vals_v, out_hbm.at[pl.ds(base, per_w)])

    return gather_k


def _sc_stats_build(B, C):
    info = plsc.get_sparse_core_info()
    nw = info.num_cores * info.num_subcores
    rpt = B // nw
    nch = 2
    chunk = C // nch
    u = 5
    nit = chunk // (16 * u)
    assert chunk % (16 * u) == 0
    mesh = plsc.VectorSubcoreMesh(core_axis_name="c", subcore_axis_name="s")
    neg_inf = float("-inf")

    @functools.partial(
        pl.kernel,
        mesh=mesh,
        out_type=jax.ShapeDtypeStruct((B * 48,), jnp.float32),
        scratch_types=[
            pltpu.VMEM((chunk,), jnp.float32),
            pltpu.VMEM((chunk,), jnp.float32),
            pltpu.VMEM((rpt * 48,), jnp.float32),
            pltpu.SemaphoreType.DMA,
            pltpu.SemaphoreType.DMA,
        ],
    )
    def stats_k(pred_hbm, out_hbm, buf0, buf1, stage, sem0, sem1):
        wid = lax.axis_index("s") * info.num_cores + lax.axis_index("c")
        row0 = wid * rpt
        bufs = (buf0, buf1)
        sems = (sem0, sem1)

        def dma(r, p):
            off = (row0 + r) * C + p * chunk
            return pltpu.make_async_copy(
                pred_hbm.at[pl.ds(off, chunk)], bufs[p], sems[p]
            )

        def chunk_stats(buf):
            def pass_a(k, carry):
                vm, ts = carry
                for uu in range(u):
                    v = buf[pl.ds((k * u + uu) * 16, 16)]
                    vm = jnp.maximum(vm, v)
                    ts = ts + v
                return vm, ts

            vm0 = jnp.full((16,), neg_inf, jnp.float32)
            z = jnp.zeros((16,), jnp.float32)
            vm, ts = lax.fori_loop(0, nit, pass_a, (vm0, z))

            def pass_b(k, s):
                for uu in range(u):
                    v = buf[pl.ds((k * u + uu) * 16, 16)]
                    s = s + jnp.exp(v - vm)
                return s

            s = lax.fori_loop(0, nit, pass_b, z)
            return vm, s, ts

        dma(0, 0).start()
        dma(0, 1).start()

        def row_body(r, dummy):
            dma(r, 0).wait()
            vm_a, s_a, ts_a = chunk_stats(buf0)

            @pl.when(r + 1 < rpt)
            def _():
                dma(r + 1, 0).start()

            dma(r, 1).wait()
            vm_b, s_b, ts_b = chunk_stats(buf1)

            @pl.when(r + 1 < rpt)
            def _():
                dma(r + 1, 1).start()

            vm = jnp.maximum(vm_a, vm_b)
            s = s_a * jnp.exp(vm_a - vm) + s_b * jnp.exp(vm_b - vm)
            ts = ts_a + ts_b
            stage[pl.ds(r * 48, 16)] = vm
            stage[pl.ds(r * 48 + 16, 16)] = s
            stage[pl.ds(r * 48 + 32, 16)] = ts
            return dummy

        lax.fori_loop(0, rpt, row_body, jnp.int32(0))
        pltpu.sync_copy(stage, out_hbm.at[pl.ds(row0 * 48, rpt * 48)])

    return stats_k


def _tc_finish_build(B, C):
    eps = SMOOTHING / (C - 1)

    def body(stats_ref, vals_ref, out_ref):
        st = stats_ref[...]
        vm = st[:, 0:16]
        s16 = st[:, 16:32]
        ts16 = st[:, 32:48]
        m = jnp.max(vm, axis=1, keepdims=True)
        s = jnp.sum(s16 * jnp.exp(vm - m), axis=1, keepdims=True)
        t = jnp.sum(ts16, axis=1, keepdims=True)
        lse = m + jnp.log(s)
        loss = eps * (C * lse - t) + (CONFIDENCE - eps) * (lse - vals_ref[...])
        out_ref[...] = jnp.sum(loss, axis=(0, 1), keepdims=True) * (1.0 / B)

    return pl.pallas_call(
        body,
        out_shape=jax.ShapeDtypeStruct((1, 1), jnp.float32),
    )


def _tc_main_build(B, C):
    eps = SMOOTHING / (C - 1)
    nchunks = B // BR

    def body(pred_hbm, vals_ref, out_ref, buf_ref, sem_ref):
        def dma(ci, slot):
            return pltpu.make_async_copy(
                pred_hbm.at[pl.ds(ci * BR, BR), :],
                buf_ref.at[slot],
                sem_ref.at[slot],
            )

        for b in range(NBUF):
            dma(b, b).start(priority=b % 2)

        def step(pi, acc):
            for par in (0, 1):
                ci = 2 * pi + par
                slot = lax.rem(ci, NBUF)
                dma(ci, slot).wait()
                x = buf_ref[slot]
                m = jnp.max(x, axis=1, keepdims=True)
                s = jnp.sum(jnp.exp(x - m), axis=1, keepdims=True)
                t = jnp.sum(x, axis=1, keepdims=True)
                lse = m + jnp.log(s)
                pt = vals_ref[pl.ds(ci * BR, BR), :]
                loss = eps * (C * lse - t) + (CONFIDENCE - eps) * (lse - pt)

                @pl.when(ci + NBUF < nchunks)
                def _():
                    dma(ci + NBUF, slot).start(priority=par)

                acc = acc + jnp.sum(loss)
            return acc

        acc = lax.fori_loop(0, nchunks // 2, step, jnp.float32(0.0))
        out_ref[...] = jnp.full((1, 1), acc * (1.0 / B), jnp.float32)

    return pl.pallas_call(
        body,
        in_specs=[
            pl.BlockSpec(memory_space=pl.ANY),
            pl.BlockSpec(memory_space=pltpu.VMEM),
        ],
        out_specs=pl.BlockSpec(memory_space=pltpu.VMEM),
        out_shape=jax.ShapeDtypeStruct((1, 1), jnp.float32),
        scratch_shapes=[
            pltpu.VMEM((NBUF, BR, C), jnp.float32),
            pltpu.SemaphoreType.DMA((NBUF,)),
        ],
    )


def kernel(pred, target):
    B, C = pred.shape
    pred_flat = pred.reshape(-1)
    vals = _sc_gather_build(B, C)(pred_flat, target.astype(jnp.int32))
    stats = _sc_stats_build(B, C)(pred_flat)
    out = _tc_finish_build(B, C)(stats.reshape(B, 48), vals.reshape(B, 1))
    return out[0, 0]


# speedup vs baseline: 1.0290x; 1.0290x over previous
import functools

import jax
import jax.numpy as jnp
from jax import lax
from jax.experimental import pallas as pl
from jax.experimental.pallas import tpu as pltpu
from jax.experimental.pallas import tpu_sc as plsc

SMOOTHING = 0.1
CONFIDENCE = 1.0 - SMOOTHING

BR = 8
NBUF = 8


def _sc_gather_build(B, C):
    info = plsc.get_sparse_core_info()
    nw = info.num_cores * info.num_subcores
    per_w = B // nw
    mesh = plsc.VectorSubcoreMesh(core_axis_name="c", subcore_axis_name="s")

    @functools.partial(
        pl.kernel,
        mesh=mesh,
        out_type=jax.ShapeDtypeStruct((B,), jnp.float32),
        scratch_types=[
            pltpu.VMEM((per_w,), jnp.int32),
            pltpu.VMEM((per_w,), jnp.float32),
            pltpu.SemaphoreType.DMA,
        ],
    )
    def gather_k(pred_flat_hbm, target_hbm, out_hbm, idx_v, vals_v, sem):
        wid = lax.axis_index("s") * info.num_cores + lax.axis_index("c")
        base = wid * per_w
        pltpu.sync_copy(target_hbm.at[pl.ds(base, per_w)], idx_v)
        for jj in range(per_w // 16):
            t = idx_v[pl.ds(jj * 16, 16)]
            rows = (base + jj * 16) + lax.iota(jnp.int32, 16)
            idx_v[pl.ds(jj * 16, 16)] = t + rows * C
        pltpu.async_copy(pred_flat_hbm.at[idx_v], vals_v, sem).wait()
        pltpu.sync_copy(vals_v, out_hbm.at[pl.ds(base, per_w)])

    return gather_k


def _sc_stats_build(B, C):
    info = plsc.get_sparse_core_info()
    nw = info.num_cores * info.num_subcores
    rpt = B // nw
    nch = 2
    chunk = C // nch
    u = 25
    nit = chunk // (16 * u)
    assert chunk % (16 * u) == 0
    mesh = plsc.VectorSubcoreMesh(core_axis_name="c", subcore_axis_name="s")
    neg_inf = float("-inf")

    @functools.partial(
        pl.kernel,
        mesh=mesh,
        out_type=jax.ShapeDtypeStruct((B * 48,), jnp.float32),
        scratch_types=[
            pltpu.VMEM((chunk,), jnp.float32),
            pltpu.VMEM((chunk,), jnp.float32),
            pltpu.VMEM((rpt * 48,), jnp.float32),
            pltpu.SemaphoreType.DMA,
            pltpu.SemaphoreType.DMA,
        ],
    )
    def stats_k(pred_hbm, out_hbm, buf0, buf1, stage, sem0, sem1):
        wid = lax.axis_index("s") * info.num_cores + lax.axis_index("c")
        row0 = wid * rpt
        bufs = (buf0, buf1)
        sems = (sem0, sem1)

        def dma(r, p):
            off = (row0 + r) * C + p * chunk
            return pltpu.make_async_copy(
                pred_hbm.at[pl.ds(off, chunk)], bufs[p], sems[p]
            )

        def chunk_stats(buf):
            def pass_a(k, carry):
                vm, ts = carry
                for uu in range(u):
                    v = buf[pl.ds((k * u + uu) * 16, 16)]
                    vm = jnp.maximum(vm, v)
                    ts = ts + v
                return vm, ts

            vm0 = jnp.full((16,), neg_inf, jnp.float32)
            z = jnp.zeros((16,), jnp.float32)
            vm, ts = lax.fori_loop(0, nit, pass_a, (vm0, z))

            def pass_b(k, s):
                for uu in range(u):
                    v = buf[pl.ds((k * u + uu) * 16, 16)]
                    s = s + jnp.exp(v - vm)
                return s

            s = lax.fori_loop(0, nit, pass_b, z)
            return vm, s, ts

        dma(0, 0).start()
        dma(0, 1).start()

        def row_body(r, dummy):
            dma(r, 0).wait()
            vm_a, s_a, ts_a = chunk_stats(buf0)

            @pl.when(r + 1 < rpt)
            def _():
                dma(r + 1, 0).start()

            dma(r, 1).wait()
            vm_b, s_b, ts_b = chunk_stats(buf1)

            @pl.when(r + 1 < rpt)
            def _():
                dma(r + 1, 1).start()

            vm = jnp.maximum(vm_a, vm_b)
            s = s_a * jnp.exp(vm_a - vm) + s_b * jnp.exp(vm_b - vm)
            ts = ts_a + ts_b
            stage[pl.ds(r * 48, 16)] = vm
            stage[pl.ds(r * 48 + 16, 16)] = s
            stage[pl.ds(r * 48 + 32, 16)] = ts
            return dummy

        lax.fori_loop(0, rpt, row_body, jnp.int32(0))
        pltpu.sync_copy(stage, out_hbm.at[pl.ds(row0 * 48, rpt * 48)])

    return stats_k


def _tc_finish_build(B, C):
    eps = SMOOTHING / (C - 1)

    def body(stats_ref, vals_ref, out_ref):
        st = stats_ref[...]
        vm = st[:, 0:16]
        s16 = st[:, 16:32]
        ts16 = st[:, 32:48]
        m = jnp.max(vm, axis=1, keepdims=True)
        s = jnp.sum(s16 * jnp.exp(vm - m), axis=1, keepdims=True)
        t = jnp.sum(ts16, axis=1, keepdims=True)
        lse = m + jnp.log(s)
        loss = eps * (C * lse - t) + (CONFIDENCE - eps) * (lse - vals_ref[...])
        out_ref[...] = jnp.sum(loss, axis=(0, 1), keepdims=True) * (1.0 / B)

    return pl.pallas_call(
        body,
        out_shape=jax.ShapeDtypeStruct((1, 1), jnp.float32),
    )


def _tc_main_build(B, C):
    eps = SMOOTHING / (C - 1)
    nchunks = B // BR

    def body(pred_hbm, vals_ref, out_ref, buf_ref, sem_ref):
        def dma(ci, slot):
            return pltpu.make_async_copy(
                pred_hbm.at[pl.ds(ci * BR, BR), :],
                buf_ref.at[slot],
                sem_ref.at[slot],
            )

        for b in range(NBUF):
            dma(b, b).start(priority=b % 2)

        def step(pi, acc):
            for par in (0, 1):
                ci = 2 * pi + par
                slot = lax.rem(ci, NBUF)
                dma(ci, slot).wait()
                x = buf_ref[slot]
                m = jnp.max(x, axis=1, keepdims=True)
                s = jnp.sum(jnp.exp(x - m), axis=1, keepdims=True)
                t = jnp.sum(x, axis=1, keepdims=True)
                lse = m + jnp.log(s)
                pt = vals_ref[pl.ds(ci * BR, BR), :]
                loss = eps * (C * lse - t) + (CONFIDENCE - eps) * (lse - pt)

                @pl.when(ci + NBUF < nchunks)
                def _():
                    dma(ci + NBUF, slot).start(priority=par)

                acc = acc + jnp.sum(loss)
            return acc

        acc = lax.fori_loop(0, nchunks // 2, step, jnp.float32(0.0))
        out_ref[...] = jnp.full((1, 1), acc * (1.0 / B), jnp.float32)

    return pl.pallas_call(
        body,
        in_specs=[
            pl.BlockSpec(memory_space=pl.ANY),
            pl.BlockSpec(memory_space=pltpu.VMEM),
        ],
        out_specs=pl.BlockSpec(memory_space=pltpu.VMEM),
        out_shape=jax.ShapeDtypeStruct((1, 1), jnp.float32),
        scratch_shapes=[
            pltpu.VMEM((NBUF, BR, C), jnp.float32),
            pltpu.SemaphoreType.DMA((NBUF,)),
        ],
    )


def kernel(pred, target):
    B, C = pred.shape
    pred_flat = pred.reshape(-1)
    vals = _sc_gather_build(B, C)(pred_flat, target.astype(jnp.int32))
    stats = _sc_stats_build(B, C)(pred_flat)
    out = _tc_finish_build(B, C)(stats.reshape(B, 48), vals.reshape(B, 1))
    return out[0, 0]


# speedup vs baseline: 1.1786x; 1.1454x over previous
import functools

import jax
import jax.numpy as jnp
from jax import lax
from jax.experimental import pallas as pl
from jax.experimental.pallas import tpu as pltpu
from jax.experimental.pallas import tpu_sc as plsc

SMOOTHING = 0.1
CONFIDENCE = 1.0 - SMOOTHING

BRT = 32
NSPLIT = 8
RT = 576


def _sc_gather_build(B, C):
    info = plsc.get_sparse_core_info()
    nw = info.num_cores * info.num_subcores
    per_w = B // nw
    mesh = plsc.VectorSubcoreMesh(core_axis_name="c", subcore_axis_name="s")

    @functools.partial(
        pl.kernel,
        mesh=mesh,
        out_type=jax.ShapeDtypeStruct((B,), jnp.float32),
        scratch_types=[
            pltpu.VMEM((per_w,), jnp.int32),
            pltpu.VMEM((per_w,), jnp.float32),
            pltpu.SemaphoreType.DMA,
        ],
    )
    def gather_k(pred_flat_hbm, target_hbm, out_hbm, idx_v, vals_v, sem):
        wid = lax.axis_index("s") * info.num_cores + lax.axis_index("c")
        base = wid * per_w
        pltpu.sync_copy(target_hbm.at[pl.ds(base, per_w)], idx_v)
        for jj in range(per_w // 16):
            t = idx_v[pl.ds(jj * 16, 16)]
            rows = (base + jj * 16) + lax.iota(jnp.int32, 16)
            idx_v[pl.ds(jj * 16, 16)] = t + rows * C
        pltpu.async_copy(pred_flat_hbm.at[idx_v], vals_v, sem).wait()
        pltpu.sync_copy(vals_v, out_hbm.at[pl.ds(base, per_w)])

    return gather_k


def _sc_stats_build(B, C, row_off, nrows):
    info = plsc.get_sparse_core_info()
    nw = info.num_cores * info.num_subcores
    rpt = nrows // nw
    nch = 2
    chunk = C // nch
    u = 25
    nit = chunk // (16 * u)
    assert chunk % (16 * u) == 0
    mesh = plsc.VectorSubcoreMesh(core_axis_name="c", subcore_axis_name="s")
    neg_inf = float("-inf")

    @functools.partial(
        pl.kernel,
        mesh=mesh,
        out_type=jax.ShapeDtypeStruct((nrows * 48,), jnp.float32),
        scratch_types=[
            pltpu.VMEM((chunk,), jnp.float32),
            pltpu.VMEM((chunk,), jnp.float32),
            pltpu.VMEM((rpt * 48,), jnp.float32),
            pltpu.SemaphoreType.DMA,
            pltpu.SemaphoreType.DMA,
        ],
    )
    def stats_k(pred_hbm, out_hbm, buf0, buf1, stage, sem0, sem1):
        wid = lax.axis_index("s") * info.num_cores + lax.axis_index("c")
        row0 = row_off + wid * rpt
        bufs = (buf0, buf1)
        sems = (sem0, sem1)

        def dma(r, p):
            off = (row0 + r) * C + p * chunk
            return pltpu.make_async_copy(
                pred_hbm.at[pl.ds(off, chunk)], bufs[p], sems[p]
            )

        def chunk_stats(buf):
            def pass_a(k, carry):
                vm, ts = carry
                for uu in range(u):
                    v = buf[pl.ds((k * u + uu) * 16, 16)]
                    vm = jnp.maximum(vm, v)
                    ts = ts + v
                return vm, ts

            vm0 = jnp.full((16,), neg_inf, jnp.float32)
            z = jnp.zeros((16,), jnp.float32)
            vm, ts = lax.fori_loop(0, nit, pass_a, (vm0, z))

            def pass_b(k, s):
                for uu in range(u):
                    v = buf[pl.ds((k * u + uu) * 16, 16)]
                    s = s + jnp.exp(v - vm)
                return s

            s = lax.fori_loop(0, nit, pass_b, z)
            return vm, s, ts

        dma(0, 0).start()
        dma(0, 1).start()

        def row_body(r, dummy):
            dma(r, 0).wait()
            vm_a, s_a, ts_a = chunk_stats(buf0)

            @pl.when(r + 1 < rpt)
            def _():
                dma(r + 1, 0).start()

            dma(r, 1).wait()
            vm_b, s_b, ts_b = chunk_stats(buf1)

            @pl.when(r + 1 < rpt)
            def _():
                dma(r + 1, 1).start()

            vm = jnp.maximum(vm_a, vm_b)
            s = s_a * jnp.exp(vm_a - vm) + s_b * jnp.exp(vm_b - vm)
            ts = ts_a + ts_b
            stage[pl.ds(r * 48, 16)] = vm
            stage[pl.ds(r * 48 + 16, 16)] = s
            stage[pl.ds(r * 48 + 32, 16)] = ts
            return dummy

        lax.fori_loop(0, rpt, row_body, jnp.int32(0))
        pltpu.sync_copy(stage, out_hbm.at[pl.ds(wid * rpt * 48, rpt * 48)])

    return stats_k


def _tc_finish_build(B, C):
    eps = SMOOTHING / (C - 1)

    def body(stats_ref, vals_ref, part_ref, out_ref):
        st = stats_ref[...]
        vm = st[:, 0:16]
        s16 = st[:, 16:32]
        ts16 = st[:, 32:48]
        m = jnp.max(vm, axis=1, keepdims=True)
        s = jnp.sum(s16 * jnp.exp(vm - m), axis=1, keepdims=True)
        t = jnp.sum(ts16, axis=1, keepdims=True)
        lse = m + jnp.log(s)
        loss = eps * (C * lse - t) + (CONFIDENCE - eps) * (lse - vals_ref[...])
        total = jnp.sum(loss, axis=(0, 1), keepdims=True) + part_ref[...]
        out_ref[...] = total * (1.0 / B)

    return pl.pallas_call(
        body,
        out_shape=jax.ShapeDtypeStruct((1, 1), jnp.float32),
    )


def _tc_partial_build(B, C, nrows):
    eps = SMOOTHING / (C - 1)
    nb = nrows // BRT
    ck = 128 * pl.cdiv(C, 128 * NSPLIT)
    valid_last = C - (NSPLIT - 1) * ck

    def body(*refs):
        x_refs = refs[:NSPLIT]
        vals_ref, out_ref, acc_ref = refs[NSPLIT], refs[NSPLIT + 1], refs[NSPLIT + 2]
        i = pl.program_id(0)

        @pl.when(i == 0)
        def _():
            acc_ref[...] = jnp.zeros_like(acc_ref)

        xs = [r[...] for r in x_refs]
        lanes = lax.broadcasted_iota(jnp.int32, (BRT, ck), 1)
        mask = lanes < valid_last
        xs_z = xs[:-1] + [jnp.where(mask, xs[-1], 0.0)]
        xs = xs[:-1] + [jnp.where(mask, xs[-1], -jnp.inf)]
        ms = [jnp.max(x, axis=1, keepdims=True) for x in xs]
        m = ms[0]
        for mk in ms[1:]:
            m = jnp.maximum(m, mk)
        s = jnp.zeros_like(m)
        t = jnp.zeros_like(m)
        for x, xz in zip(xs, xs_z):
            s += jnp.sum(jnp.exp(x - m), axis=1, keepdims=True)
            t += jnp.sum(xz, axis=1, keepdims=True)
        lse = m + jnp.log(s)
        loss = eps * (C * lse - t) + (CONFIDENCE - eps) * (lse - vals_ref[...])
        acc_ref[...] += jnp.sum(loss, axis=(0, 1), keepdims=True)

        @pl.when(i == nb - 1)
        def _():
            out_ref[...] = acc_ref[...]

    return pl.pallas_call(
        body,
        grid=(nb,),
        in_specs=[
            pl.BlockSpec((BRT, ck), lambda i, kk=k: (i, kk)) for k in range(NSPLIT)
        ]
        + [pl.BlockSpec((BRT, 1), lambda i: (i, 0))],
        out_specs=pl.BlockSpec((1, 1), lambda i: (0, 0)),
        out_shape=jax.ShapeDtypeStruct((1, 1), jnp.float32),
        scratch_shapes=[
            pltpu.VMEM((1, 1), jnp.float32),
        ],
        compiler_params=pltpu.CompilerParams(
            dimension_semantics=("arbitrary",),
        ),
    )


def kernel(pred, target):
    B, C = pred.shape
    pred_flat = pred.reshape(-1)
    vals = _sc_gather_build(B, C)(pred_flat, target.astype(jnp.int32))
    vals2 = vals.reshape(B, 1)
    stats = _sc_stats_build(B, C, RT, B - RT)(pred_flat)
    part = _tc_partial_build(B, C, RT)(*([pred] * NSPLIT), vals2[:RT])
    out = _tc_finish_build(B, C)(
        stats.reshape(B - RT, 48), vals2[RT:], part
    )
    return out[0, 0]


# speedup vs baseline: 1.2636x; 1.0721x over previous
import functools

import jax
import jax.numpy as jnp
from jax import lax
from jax.experimental import pallas as pl
from jax.experimental.pallas import tpu as pltpu
from jax.experimental.pallas import tpu_sc as plsc

SMOOTHING = 0.1
CONFIDENCE = 1.0 - SMOOTHING

BRT = 32
NSPLIT = 8


def _sc_gather_build(B, C):
    info = plsc.get_sparse_core_info()
    nw = info.num_cores * info.num_subcores
    per_w = B // nw
    mesh = plsc.VectorSubcoreMesh(core_axis_name="c", subcore_axis_name="s")

    @functools.partial(
        pl.kernel,
        mesh=mesh,
        out_type=jax.ShapeDtypeStruct((B,), jnp.float32),
        scratch_types=[
            pltpu.VMEM((per_w,), jnp.int32),
            pltpu.VMEM((per_w,), jnp.float32),
            pltpu.SemaphoreType.DMA,
        ],
    )
    def gather_k(pred_flat_hbm, target_hbm, out_hbm, idx_v, vals_v, sem):
        wid = lax.axis_index("s") * info.num_cores + lax.axis_index("c")
        base = wid * per_w
        pltpu.sync_copy(target_hbm.at[pl.ds(base, per_w)], idx_v)
        for jj in range(per_w // 16):
            t = idx_v[pl.ds(jj * 16, 16)]
            rows = (base + jj * 16) + lax.iota(jnp.int32, 16)
            idx_v[pl.ds(jj * 16, 16)] = t + rows * C
        pltpu.async_copy(pred_flat_hbm.at[idx_v], vals_v, sem).wait()
        pltpu.sync_copy(vals_v, out_hbm.at[pl.ds(base, per_w)])

    return gather_k


def _tc_partial_build(B, C, nrows):
    eps = SMOOTHING / (C - 1)
    nb = nrows // BRT
    ck = 128 * pl.cdiv(C, 128 * NSPLIT)
    valid_last = C - (NSPLIT - 1) * ck

    def body(*refs):
        x_refs = refs[:NSPLIT]
        vals_ref, out_ref, acc_ref = refs[NSPLIT], refs[NSPLIT + 1], refs[NSPLIT + 2]
        i = pl.program_id(0)

        @pl.when(i == 0)
        def _():
            acc_ref[...] = jnp.zeros_like(acc_ref)

        xs = [r[...] for r in x_refs]
        lanes = lax.broadcasted_iota(jnp.int32, (BRT, ck), 1)
        mask = lanes < valid_last
        xs_z = xs[:-1] + [jnp.where(mask, xs[-1], 0.0)]
        xs = xs[:-1] + [jnp.where(mask, xs[-1], -jnp.inf)]
        ms = [jnp.max(x, axis=1, keepdims=True) for x in xs]
        m = ms[0]
        for mk in ms[1:]:
            m = jnp.maximum(m, mk)
        s = jnp.zeros_like(m)
        t = jnp.zeros_like(m)
        for x, xz in zip(xs, xs_z):
            s += jnp.sum(jnp.exp(x - m), axis=1, keepdims=True)
            t += jnp.sum(xz, axis=1, keepdims=True)
        lse = m + jnp.log(s)
        loss = eps * (C * lse - t) + (CONFIDENCE - eps) * (lse - vals_ref[...])
        acc_ref[...] += jnp.sum(loss, axis=(0, 1), keepdims=True)

        @pl.when(i == nb - 1)
        def _():
            out_ref[...] = acc_ref[...] * (1.0 / B)

    return pl.pallas_call(
        body,
        grid=(nb,),
        in_specs=[
            pl.BlockSpec((BRT, ck), lambda i, kk=k: (i, kk)) for k in range(NSPLIT)
        ]
        + [pl.BlockSpec((BRT, 1), lambda i: (i, 0))],
        out_specs=pl.BlockSpec((1, 1), lambda i: (0, 0)),
        out_shape=jax.ShapeDtypeStruct((1, 1), jnp.float32),
        scratch_shapes=[
            pltpu.VMEM((1, 1), jnp.float32),
        ],
        compiler_params=pltpu.CompilerParams(
            dimension_semantics=("arbitrary",),
        ),
    )


def kernel(pred, target):
    B, C = pred.shape
    pred_flat = pred.reshape(-1)
    vals = _sc_gather_build(B, C)(pred_flat, target.astype(jnp.int32))
    out = _tc_partial_build(B, C, B)(*([pred] * NSPLIT), vals.reshape(B, 1))
    return out[0, 0]
